# Initial kernel scaffold; baseline (speedup 1.0000x reference)
import jax
import jax.numpy as jnp
from jax.experimental import pallas as pl


def kernel(test, question, tag, correct, mask, interaction, dffclt, dscrmn, gussng, testTag, user_correct_answer, user_total_answer, user_acc, user_mean, assessment_mean, test_mean, knowledgeTag_mean, time_to_solve, prior_testTag_frequency, emb_interaction, emb_test, emb_question, emb_tag, emb_testTag, lin_W, lin_b, comb_W, comb_b):
    raise NotImplementedError("write your pallas kernel here")


# speedup vs baseline: 8.5760x; 8.5760x over previous
import functools

import jax
import jax.numpy as jnp
from jax import lax
from jax.experimental import pallas as pl
from jax.experimental.pallas import tpu as pltpu
from jax.experimental.pallas import tpu_sc as plsc

B, S = 1024, 200
NPOS = B * S
HD = 64
INTD = 21
NC, NS = 2, 16
NW = NC * NS
PER_W = NPOS // NW
CH = 320
NCHUNK = PER_W // CH

_f32 = jnp.float32


def _proj_body(et, eq, eg, e3, e10, w_ref, pt, pq, pg, p3, p10):
    W = w_ref[...]
    dot = functools.partial(jnp.dot, preferred_element_type=_f32)
    pt[...] = dot(et[...], W[21:42])
    pq[...] = dot(eq[...], W[42:63])
    pg[...] = dot(eg[...], W[63:84])
    p3[...] = dot(e3[...], W[0:21])
    p10[...] = dot(e10[...], W[99:104][:5])


def _project_tables(emb_test, emb_question, emb_tag, emb_int, emb_tt, comb_W):
    n_t, n_q, n_g = emb_test.shape[0], emb_question.shape[0], emb_tag.shape[0]
    out_shapes = (
        jax.ShapeDtypeStruct((n_t, HD), _f32),
        jax.ShapeDtypeStruct((n_q, HD), _f32),
        jax.ShapeDtypeStruct((n_g, HD), _f32),
        jax.ShapeDtypeStruct((3, HD), _f32),
        jax.ShapeDtypeStruct((10, HD), _f32),
    )
    return pl.pallas_call(_proj_body, out_shape=out_shapes)(
        emb_test, emb_question, emb_tag, emb_int, emb_tt, comb_W)


def _gather_body(tidx_hbm, qidx_hbm, gidx_hbm, pt_hbm, pq_hbm, pg_hbm,
                 ident_hbm, out_hbm,
                 idx_t, idx_q, idx_g, tmp1, tmp2, tmp3, ident_v, acc_sh,
                 sem1, sem2, sem3):
    c = lax.axis_index("c")
    s = lax.axis_index("s")
    wid = s * NC + c
    base = wid * PER_W

    pltpu.sync_copy(ident_hbm, ident_v)
    off = (s * CH).astype(jnp.int32)
    for k in range(CH // 16):
        sl = pl.ds(k * 16, 16)
        ident_v[sl] = ident_v[sl] + off

    def chunk(i, carry):
        pos = base + i * CH
        pltpu.sync_copy(tidx_hbm.at[pl.ds(pos, CH)], idx_t)
        pltpu.sync_copy(qidx_hbm.at[pl.ds(pos, CH)], idx_q)
        pltpu.sync_copy(gidx_hbm.at[pl.ds(pos, CH)], idx_g)
        cp1 = pltpu.async_copy(pt_hbm.at[idx_t], tmp1, sem1)
        cp2 = pltpu.async_copy(pq_hbm.at[idx_q], tmp2, sem2)
        cp3 = pltpu.async_copy(pg_hbm.at[idx_g], tmp3, sem3)
        cp1.wait()
        cp2.wait()
        cp3.wait()
        mine = acc_sh.at[pl.ds(s * CH, CH)]
        pltpu.sync_copy(tmp1, mine)
        pltpu.sync_copy(tmp2, acc_sh.at[ident_v], add=True)
        pltpu.sync_copy(tmp3, acc_sh.at[ident_v], add=True)
        pltpu.sync_copy(mine, out_hbm.at[pl.ds(pos, CH)])
        return carry

    lax.fori_loop(0, NCHUNK, chunk, 0)


def _gather_sum(tidx, qidx, gidx, pt, pq, pg):
    mesh = plsc.VectorSubcoreMesh(core_axis_name="c", subcore_axis_name="s")
    ident = jnp.arange(CH, dtype=jnp.int32)
    fn = pl.kernel(
        _gather_body,
        out_type=jax.ShapeDtypeStruct((NPOS, HD), _f32),
        mesh=mesh,
        compiler_params=pltpu.CompilerParams(use_tc_tiling_on_sc=False),
        scratch_types=[
            pltpu.VMEM((CH,), jnp.int32),
            pltpu.VMEM((CH,), jnp.int32),
            pltpu.VMEM((CH,), jnp.int32),
            pltpu.VMEM((CH, HD), _f32),
            pltpu.VMEM((CH, HD), _f32),
            pltpu.VMEM((CH, HD), _f32),
            pltpu.VMEM((CH,), jnp.int32),
            pltpu.VMEM_SHARED((NS * CH, HD), _f32),
            pltpu.SemaphoreType.DMA,
            pltpu.SemaphoreType.DMA,
            pltpu.SemaphoreType.DMA,
        ],
    )
    return fn(tidx, qidx, gidx, pt, pq, pg, ident)


CHC = 2048


def _dense_body(x_ref, gsum_ref, g_ref, h_ref, w60_ref, p3_ref, p10_ref,
                b_ref, out_ref):
    x = x_ref[...]
    dot = functools.partial(jnp.dot, preferred_element_type=_f32)
    a = jax.nn.sigmoid(dot(x, g_ref[...]) + h_ref[...])
    o = dot(a, w60_ref[...]) + gsum_ref[...] + b_ref[...]
    iv = x[:, 12:13]
    tv = x[:, 13:14]
    i3 = lax.broadcasted_iota(jnp.int32, (CHC, 3), 1).astype(_f32)
    i10 = lax.broadcasted_iota(jnp.int32, (CHC, 10), 1).astype(_f32)
    oh3 = jnp.where(iv == i3, 1.0, 0.0).astype(_f32)
    oh10 = jnp.where(tv == i10, 1.0, 0.0).astype(_f32)
    out_ref[...] = o + dot(oh3, p3_ref[...]) + dot(oh10, p10_ref[...])


def _dense_part(xc, gsum, G, h, W60, p3, p10, bias):
    grid = (NPOS // CHC,)
    fixed = lambda i: (0, 0)
    return pl.pallas_call(
        _dense_body,
        grid=grid,
        in_specs=[
            pl.BlockSpec((CHC, 14), lambda i: (i, 0)),
            pl.BlockSpec((CHC, HD), lambda i: (i, 0)),
            pl.BlockSpec((14, 60), fixed),
            pl.BlockSpec((1, 60), fixed),
            pl.BlockSpec((60, HD), fixed),
            pl.BlockSpec((3, HD), fixed),
            pl.BlockSpec((10, HD), fixed),
            pl.BlockSpec((1, HD), fixed),
        ],
        out_specs=pl.BlockSpec((CHC, HD), lambda i: (i, 0)),
        out_shape=jax.ShapeDtypeStruct((NPOS, HD), _f32),
    )(xc, gsum, G, h, W60, p3, p10, bias)


def kernel(test, question, tag, correct, mask, interaction, dffclt, dscrmn,
           gussng, testTag, user_correct_answer, user_total_answer, user_acc,
           user_mean, assessment_mean, test_mean, knowledgeTag_mean,
           time_to_solve, prior_testTag_frequency, emb_interaction, emb_test,
           emb_question, emb_tag, emb_testTag, lin_W, lin_b, comb_W, comb_b):
    tidx = test.reshape(NPOS).astype(jnp.int32)
    qidx = question.reshape(NPOS).astype(jnp.int32)
    gidx = tag.reshape(NPOS).astype(jnp.int32)

    conti = [dffclt, dscrmn, gussng, user_correct_answer, user_total_answer,
             user_acc, user_mean, assessment_mean, test_mean,
             knowledgeTag_mean, time_to_solve, prior_testTag_frequency]
    xc = jnp.stack(
        conti + [interaction.astype(_f32), testTag.astype(_f32)],
        axis=-1).reshape(NPOS, 14)

    G = jnp.concatenate(
        [jnp.kron(jnp.eye(12, dtype=_f32), lin_W[0:1]),
         jnp.zeros((2, 60), _f32)], axis=0)
    h = jnp.tile(lin_b, 12).reshape(1, 60)
    W60 = jnp.concatenate([comb_W[84:99], comb_W[104:149]], axis=0)

    pt, pq, pg, p3, p10 = _project_tables(
        emb_test, emb_question, emb_tag, emb_interaction, emb_testTag, comb_W)
    gsum = _gather_sum(tidx, qidx, gidx, pt, pq, pg)
    out = _dense_part(xc, gsum, G, h, W60, p3, p10, comb_b.reshape(1, HD))
    return out.reshape(B, S, HD)


# speedup vs baseline: 8.7980x; 1.0259x over previous
import functools

import jax
import jax.numpy as jnp
from jax import lax
from jax.experimental import pallas as pl
from jax.experimental.pallas import tpu as pltpu
from jax.experimental.pallas import tpu_sc as plsc

B, S = 1024, 200
NPOS = B * S
HD = 64
INTD = 21
NC, NS = 2, 16
NW = NC * NS
PER_W = NPOS // NW
CH = 160
NCHUNK = PER_W // CH

_f32 = jnp.float32


def _proj_body(et, eq, eg, e3, e10, w_ref, pt, pq, pg, p3, p10):
    W = w_ref[...]
    dot = functools.partial(jnp.dot, preferred_element_type=_f32)
    pt[...] = dot(et[...], W[21:42])
    pq[...] = dot(eq[...], W[42:63])
    pg[...] = dot(eg[...], W[63:84])
    p3[...] = dot(e3[...], W[0:21])
    p10[...] = dot(e10[...], W[99:104][:5])


def _project_tables(emb_test, emb_question, emb_tag, emb_int, emb_tt, comb_W):
    n_t, n_q, n_g = emb_test.shape[0], emb_question.shape[0], emb_tag.shape[0]
    out_shapes = (
        jax.ShapeDtypeStruct((n_t, HD), _f32),
        jax.ShapeDtypeStruct((n_q, HD), _f32),
        jax.ShapeDtypeStruct((n_g, HD), _f32),
        jax.ShapeDtypeStruct((3, HD), _f32),
        jax.ShapeDtypeStruct((10, HD), _f32),
    )
    return pl.pallas_call(_proj_body, out_shape=out_shapes)(
        emb_test, emb_question, emb_tag, emb_int, emb_tt, comb_W)


def _issue_gathers(pt_hbm, pq_hbm, pg_hbm, ti_v, qi_v, gi_v, tmp, i, b, sem):
    pltpu.async_copy(pt_hbm.at[ti_v.at[i]], tmp.at[b, 0], sem)
    pltpu.async_copy(pq_hbm.at[qi_v.at[i]], tmp.at[b, 1], sem)
    pltpu.async_copy(pg_hbm.at[gi_v.at[i]], tmp.at[b, 2], sem)


def _drain_gathers(pt_hbm, tmp, b, sem):
    for j in range(3):
        pltpu.make_async_copy(pt_hbm.at[pl.ds(0, CH)], tmp.at[b, j], sem).wait()


def _gather_body(tidx_hbm, qidx_hbm, gidx_hbm, pt_hbm, pq_hbm, pg_hbm,
                 ident_hbm, out_hbm,
                 ti_v, qi_v, gi_v, ident_v, tmp, acc_sh, sem0, sem1):
    c = lax.axis_index("c")
    s = lax.axis_index("s")
    wid = s * NC + c
    base = wid * PER_W

    pltpu.sync_copy(tidx_hbm.at[wid], ti_v)
    pltpu.sync_copy(qidx_hbm.at[wid], qi_v)
    pltpu.sync_copy(gidx_hbm.at[wid], gi_v)
    pltpu.sync_copy(ident_hbm, ident_v)
    off = s * CH
    for k in range(CH // 16):
        sl = pl.ds(k * 16, 16)
        ident_v[sl] = ident_v[sl] + off

    _issue_gathers(pt_hbm, pq_hbm, pg_hbm, ti_v, qi_v, gi_v, tmp, 0, 0, sem0)

    sems = (sem0, sem1)

    def chunk(i, b):
        @pl.when(i < NCHUNK - 1)
        def _():
            _issue_gathers(pt_hbm, pq_hbm, pg_hbm, ti_v, qi_v, gi_v, tmp,
                           i + 1, 1 - b, sems[1 - b])
        _drain_gathers(pt_hbm, tmp, b, sems[b])
        mine = acc_sh.at[pl.ds(s * CH, CH)]
        pltpu.sync_copy(tmp.at[b, 0], mine)
        pltpu.sync_copy(tmp.at[b, 1], acc_sh.at[ident_v], add=True)
        pltpu.sync_copy(tmp.at[b, 2], acc_sh.at[ident_v], add=True)
        pltpu.sync_copy(mine, out_hbm.at[pl.ds(base + i * CH, CH)])

    def two_chunks(k, carry):
        chunk(2 * k, 0)
        chunk(2 * k + 1, 1)
        return carry

    lax.fori_loop(0, NCHUNK // 2, two_chunks, 0)
    if NCHUNK % 2:
        chunk(NCHUNK - 1, 0)


def _gather_sum(tidx, qidx, gidx, pt, pq, pg):
    mesh = plsc.VectorSubcoreMesh(core_axis_name="c", subcore_axis_name="s")
    ident = jnp.arange(CH, dtype=jnp.int32)
    fn = pl.kernel(
        _gather_body,
        out_type=jax.ShapeDtypeStruct((NPOS, HD), _f32),
        mesh=mesh,
        compiler_params=pltpu.CompilerParams(use_tc_tiling_on_sc=False),
        scratch_types=[
            pltpu.VMEM((NCHUNK, CH), jnp.int32),
            pltpu.VMEM((NCHUNK, CH), jnp.int32),
            pltpu.VMEM((NCHUNK, CH), jnp.int32),
            pltpu.VMEM((CH,), jnp.int32),
            pltpu.VMEM((2, 3, CH, HD), _f32),
            pltpu.VMEM_SHARED((NS * CH, HD), _f32),
            pltpu.SemaphoreType.DMA,
            pltpu.SemaphoreType.DMA,
        ],
    )
    t3 = tidx.reshape(NW, NCHUNK, CH)
    q3 = qidx.reshape(NW, NCHUNK, CH)
    g3 = gidx.reshape(NW, NCHUNK, CH)
    return fn(t3, q3, g3, pt, pq, pg, ident)


CHC = 2048


def _dense_body(x_ref, gsum_ref, g_ref, h_ref, w60_ref, p3_ref, p10_ref,
                b_ref, out_ref):
    x = x_ref[...]
    dot = functools.partial(jnp.dot, preferred_element_type=_f32)
    a = jax.nn.sigmoid(dot(x, g_ref[...]) + h_ref[...])
    o = dot(a, w60_ref[...]) + gsum_ref[...] + b_ref[...]
    iv = x[:, 12:13]
    tv = x[:, 13:14]
    i3 = lax.broadcasted_iota(jnp.int32, (CHC, 3), 1).astype(_f32)
    i10 = lax.broadcasted_iota(jnp.int32, (CHC, 10), 1).astype(_f32)
    oh3 = jnp.where(iv == i3, 1.0, 0.0).astype(_f32)
    oh10 = jnp.where(tv == i10, 1.0, 0.0).astype(_f32)
    out_ref[...] = o + dot(oh3, p3_ref[...]) + dot(oh10, p10_ref[...])


def _dense_part(xc, gsum, G, h, W60, p3, p10, bias):
    grid = (NPOS // CHC,)
    fixed = lambda i: (0, 0)
    return pl.pallas_call(
        _dense_body,
        grid=grid,
        in_specs=[
            pl.BlockSpec((CHC, 14), lambda i: (i, 0)),
            pl.BlockSpec((CHC, HD), lambda i: (i, 0)),
            pl.BlockSpec((14, 60), fixed),
            pl.BlockSpec((1, 60), fixed),
            pl.BlockSpec((60, HD), fixed),
            pl.BlockSpec((3, HD), fixed),
            pl.BlockSpec((10, HD), fixed),
            pl.BlockSpec((1, HD), fixed),
        ],
        out_specs=pl.BlockSpec((CHC, HD), lambda i: (i, 0)),
        out_shape=jax.ShapeDtypeStruct((NPOS, HD), _f32),
    )(xc, gsum, G, h, W60, p3, p10, bias)


def kernel(test, question, tag, correct, mask, interaction, dffclt, dscrmn,
           gussng, testTag, user_correct_answer, user_total_answer, user_acc,
           user_mean, assessment_mean, test_mean, knowledgeTag_mean,
           time_to_solve, prior_testTag_frequency, emb_interaction, emb_test,
           emb_question, emb_tag, emb_testTag, lin_W, lin_b, comb_W, comb_b):
    tidx = test.reshape(NPOS).astype(jnp.int32)
    qidx = question.reshape(NPOS).astype(jnp.int32)
    gidx = tag.reshape(NPOS).astype(jnp.int32)

    conti = [dffclt, dscrmn, gussng, user_correct_answer, user_total_answer,
             user_acc, user_mean, assessment_mean, test_mean,
             knowledgeTag_mean, time_to_solve, prior_testTag_frequency]
    xc = jnp.stack(
        conti + [interaction.astype(_f32), testTag.astype(_f32)],
        axis=-1).reshape(NPOS, 14)

    G = jnp.concatenate(
        [jnp.kron(jnp.eye(12, dtype=_f32), lin_W[0:1]),
         jnp.zeros((2, 60), _f32)], axis=0)
    h = jnp.tile(lin_b, 12).reshape(1, 60)
    W60 = jnp.concatenate([comb_W[84:99], comb_W[104:149]], axis=0)

    pt, pq, pg, p3, p10 = _project_tables(
        emb_test, emb_question, emb_tag, emb_interaction, emb_testTag, comb_W)
    gsum = _gather_sum(tidx, qidx, gidx, pt, pq, pg)
    out = _dense_part(xc, gsum, G, h, W60, p3, p10, comb_b.reshape(1, HD))
    return out.reshape(B, S, HD)


# speedup vs baseline: 12.1832x; 1.3848x over previous
import functools

import jax
import jax.numpy as jnp
from jax import lax
from jax.experimental import pallas as pl
from jax.experimental.pallas import tpu as pltpu
from jax.experimental.pallas import tpu_sc as plsc

B, S = 1024, 200
NPOS = B * S
HD = 64
NC, NS = 2, 16
NW = NC * NS
PER_W = NPOS // NW
CH = 160
NCHUNK = PER_W // CH
NT, NQ, NG = 1540, 9456, 914

_f32 = jnp.float32
_dot = functools.partial(jnp.dot, preferred_element_type=_f32)
_DNT = (((0,), (0,)), ((), ()))


def _proj_body(et2, eq2, eg2, e3, e10, wt2, wq2, wg2, w3, w10,
               pt2, pq2, pg2, p3, p10):
    pt2[...] = _dot(et2[...], wt2[...])
    pq2[...] = _dot(eq2[...], wq2[...])
    pg2[...] = _dot(eg2[...], wg2[...])
    p3[...] = _dot(e3[...], w3[...])
    p10[...] = _dot(e10[...], w10[...])


def _project_tables(emb_test, emb_question, emb_tag, emb_int, emb_tt, comb_W):
    et2 = jnp.concatenate(
        [emb_test, jnp.zeros((1, 21), _f32)], axis=0).reshape(NT // 2, 42)
    eq2 = emb_question.reshape(NQ // 2, 42)
    eg2 = emb_tag.reshape(NG // 2, 42)

    def blockdiag(w):
        z = jnp.zeros((21, HD), _f32)
        return jnp.concatenate(
            [jnp.concatenate([w, z], axis=1),
             jnp.concatenate([z, w], axis=1)], axis=0)

    wt2 = blockdiag(comb_W[21:42])
    wq2 = blockdiag(comb_W[42:63])
    wg2 = blockdiag(comb_W[63:84])

    out_shapes = (
        jax.ShapeDtypeStruct((NT // 2, 128), _f32),
        jax.ShapeDtypeStruct((NQ // 2, 128), _f32),
        jax.ShapeDtypeStruct((NG // 2, 128), _f32),
        jax.ShapeDtypeStruct((3, HD), _f32),
        jax.ShapeDtypeStruct((10, HD), _f32),
    )
    pt2, pq2, pg2, p3, p10 = pl.pallas_call(_proj_body, out_shape=out_shapes)(
        et2, eq2, eg2, emb_int, emb_tt, wt2, wq2, wg2,
        comb_W[0:21], comb_W[99:104])
    return (pt2.reshape(NT, HD), pq2.reshape(NQ, HD), pg2.reshape(NG, HD),
            p3, p10)


def _issue_gathers(pt_hbm, pq_hbm, pg_hbm, ti_v, qi_v, gi_v, tmp, i, b, sem):
    sl = pl.ds(i * CH, CH)
    pltpu.async_copy(pt_hbm.at[ti_v.at[sl]], tmp.at[b, 0], sem)
    pltpu.async_copy(pq_hbm.at[qi_v.at[sl]], tmp.at[b, 1], sem)
    pltpu.async_copy(pg_hbm.at[gi_v.at[sl]], tmp.at[b, 2], sem)


def _drain_gathers(pt_hbm, tmp, b, sem):
    for j in range(3):
        pltpu.make_async_copy(pt_hbm.at[pl.ds(0, CH)], tmp.at[b, j], sem).wait()


def _gather_body(tidx_hbm, qidx_hbm, gidx_hbm, pt_hbm, pq_hbm, pg_hbm,
                 ident_hbm, out_hbm,
                 ti_v, qi_v, gi_v, ident_v, tmp, acc_sh, sem0, sem1):
    c = lax.axis_index("c")
    s = lax.axis_index("s")
    wid = s * NC + c
    base = wid * PER_W

    pltpu.sync_copy(tidx_hbm.at[pl.ds(base, PER_W)], ti_v)
    pltpu.sync_copy(qidx_hbm.at[pl.ds(base, PER_W)], qi_v)
    pltpu.sync_copy(gidx_hbm.at[pl.ds(base, PER_W)], gi_v)
    pltpu.sync_copy(ident_hbm, ident_v)
    off = s * CH
    for k in range(CH // 16):
        sl = pl.ds(k * 16, 16)
        ident_v[sl] = ident_v[sl] + off

    _issue_gathers(pt_hbm, pq_hbm, pg_hbm, ti_v, qi_v, gi_v, tmp, 0, 0, sem0)

    sems = (sem0, sem1)

    def chunk(i, b):
        @pl.when(i < NCHUNK - 1)
        def _():
            _issue_gathers(pt_hbm, pq_hbm, pg_hbm, ti_v, qi_v, gi_v, tmp,
                           i + 1, 1 - b, sems[1 - b])
        _drain_gathers(pt_hbm, tmp, b, sems[b])
        mine = acc_sh.at[pl.ds(s * CH, CH)]
        pltpu.sync_copy(tmp.at[b, 0], mine)
        pltpu.sync_copy(tmp.at[b, 1], acc_sh.at[ident_v], add=True)
        pltpu.sync_copy(tmp.at[b, 2], acc_sh.at[ident_v], add=True)
        pltpu.sync_copy(mine, out_hbm.at[pl.ds(base + i * CH, CH),
                                         pl.ds(0, HD)])

    def two_chunks(k, carry):
        chunk(2 * k, 0)
        chunk(2 * k + 1, 1)
        return carry

    lax.fori_loop(0, NCHUNK // 2, two_chunks, 0)
    if NCHUNK % 2:
        chunk(NCHUNK - 1, 0)


def _gather_sum(tidx, qidx, gidx, pt, pq, pg):
    mesh = plsc.VectorSubcoreMesh(core_axis_name="c", subcore_axis_name="s")
    ident = jnp.arange(CH, dtype=jnp.int32)
    fn = pl.kernel(
        _gather_body,
        out_type=jax.ShapeDtypeStruct((NPOS, 128), _f32),
        mesh=mesh,
        compiler_params=pltpu.CompilerParams(use_tc_tiling_on_sc=False),
        scratch_types=[
            pltpu.VMEM((PER_W,), jnp.int32),
            pltpu.VMEM((PER_W,), jnp.int32),
            pltpu.VMEM((PER_W,), jnp.int32),
            pltpu.VMEM((CH,), jnp.int32),
            pltpu.VMEM((2, 3, CH, HD), _f32),
            pltpu.VMEM_SHARED((NS * CH, HD), _f32),
            pltpu.SemaphoreType.DMA,
            pltpu.SemaphoreType.DMA,
        ],
    )
    return fn(tidx, qidx, gidx, pt, pq, pg, ident)


CHC = 2048


def _dense_body(xt_ref, gsum_ref, g_ref, h_ref, w60_ref, p3_ref, p10_ref,
                b_ref, out_ref):
    xt = xt_ref[...]
    a = jax.nn.sigmoid(
        lax.dot_general(xt, g_ref[...], _DNT, preferred_element_type=_f32)
        + h_ref[...])
    o = _dot(a, w60_ref[...]) + b_ref[...] + gsum_ref[:, 0:HD]
    iv = xt[12:13, :]
    tv = xt[13:14, :]
    i3 = lax.broadcasted_iota(jnp.int32, (3, CHC), 0).astype(_f32)
    i10 = lax.broadcasted_iota(jnp.int32, (10, CHC), 0).astype(_f32)
    oh3 = jnp.where(iv == i3, 1.0, 0.0)
    oh10 = jnp.where(tv == i10, 1.0, 0.0)
    o = o + lax.dot_general(oh3, p3_ref[...], _DNT,
                            preferred_element_type=_f32)
    o = o + lax.dot_general(oh10, p10_ref[...], _DNT,
                            preferred_element_type=_f32)
    out_ref[...] = o


def _dense_part(xt, gsum2, G, h, W60, p3, p10, bias):
    grid = (NPOS // CHC,)
    fixed = lambda i: (0, 0)
    return pl.pallas_call(
        _dense_body,
        grid=grid,
        in_specs=[
            pl.BlockSpec((14, CHC), lambda i: (0, i)),
            pl.BlockSpec((CHC, 128), lambda i: (i, 0)),
            pl.BlockSpec((14, 60), fixed),
            pl.BlockSpec((1, 60), fixed),
            pl.BlockSpec((60, HD), fixed),
            pl.BlockSpec((3, HD), fixed),
            pl.BlockSpec((10, HD), fixed),
            pl.BlockSpec((1, HD), fixed),
        ],
        out_specs=pl.BlockSpec((CHC, HD), lambda i: (i, 0)),
        out_shape=jax.ShapeDtypeStruct((NPOS, HD), _f32),
    )(xt, gsum2, G, h, W60, p3, p10, bias)


def kernel(test, question, tag, correct, mask, interaction, dffclt, dscrmn,
           gussng, testTag, user_correct_answer, user_total_answer, user_acc,
           user_mean, assessment_mean, test_mean, knowledgeTag_mean,
           time_to_solve, prior_testTag_frequency, emb_interaction, emb_test,
           emb_question, emb_tag, emb_testTag, lin_W, lin_b, comb_W, comb_b):
    tidx = test.reshape(NPOS).astype(jnp.int32)
    qidx = question.reshape(NPOS).astype(jnp.int32)
    gidx = tag.reshape(NPOS).astype(jnp.int32)

    conti = [dffclt, dscrmn, gussng, user_correct_answer, user_total_answer,
             user_acc, user_mean, assessment_mean, test_mean,
             knowledgeTag_mean, time_to_solve, prior_testTag_frequency]
    xt = jnp.stack(
        conti + [interaction.astype(_f32), testTag.astype(_f32)],
        axis=0).reshape(14, NPOS)

    G = jnp.concatenate(
        [jnp.kron(jnp.eye(12, dtype=_f32), lin_W[0:1]),
         jnp.zeros((2, 60), _f32)], axis=0)
    h = jnp.tile(lin_b, 12).reshape(1, 60)
    W60 = jnp.concatenate([comb_W[84:99], comb_W[104:149]], axis=0)

    pt, pq, pg, p3, p10 = _project_tables(
        emb_test, emb_question, emb_tag, emb_interaction, emb_testTag, comb_W)
    gsum = _gather_sum(tidx, qidx, gidx, pt, pq, pg)
    out = _dense_part(xt, gsum, G, h, W60, p3, p10, comb_b.reshape(1, HD))
    return out.reshape(B, S, HD)


# speedup vs baseline: 13.3678x; 1.0972x over previous
import functools

import jax
import jax.numpy as jnp
from jax import lax
from jax.experimental import pallas as pl
from jax.experimental.pallas import tpu as pltpu
from jax.experimental.pallas import tpu_sc as plsc

B, S = 1024, 200
NPOS = B * S
HD = 64
NC, NS = 2, 16
NW = NC * NS
PER_W = NPOS // NW
CH = 160
NCHUNK = PER_W // CH
NT, NQ, NG = 1540, 9456, 914

_f32 = jnp.float32
_dot = functools.partial(jnp.dot, preferred_element_type=_f32)
_DNT = (((0,), (0,)), ((), ()))


def _proj_body(et2, eq2, eg2, e3, e10, wt2, wq2, wg2, w3, w10,
               pt2, pq2, pg2, p3, p10):
    pt2[...] = _dot(et2[...], wt2[...])
    pq2[...] = _dot(eq2[...], wq2[...])
    pg2[...] = _dot(eg2[...], wg2[...])
    p3[...] = _dot(e3[...], w3[...])
    p10[...] = _dot(e10[...], w10[...])


def _project_tables(emb_test, emb_question, emb_tag, emb_int, emb_tt, comb_W):
    et2 = jnp.concatenate(
        [emb_test, jnp.zeros((1, 21), _f32)], axis=0).reshape(NT // 2, 42)
    eq2 = emb_question.reshape(NQ // 2, 42)
    eg2 = emb_tag.reshape(NG // 2, 42)

    def blockdiag(w):
        z = jnp.zeros((21, HD), _f32)
        return jnp.concatenate(
            [jnp.concatenate([w, z], axis=1),
             jnp.concatenate([z, w], axis=1)], axis=0)

    wt2 = blockdiag(comb_W[21:42])
    wq2 = blockdiag(comb_W[42:63])
    wg2 = blockdiag(comb_W[63:84])

    out_shapes = (
        jax.ShapeDtypeStruct((NT // 2, 128), _f32),
        jax.ShapeDtypeStruct((NQ // 2, 128), _f32),
        jax.ShapeDtypeStruct((NG // 2, 128), _f32),
        jax.ShapeDtypeStruct((3, HD), _f32),
        jax.ShapeDtypeStruct((10, HD), _f32),
    )
    pt2, pq2, pg2, p3, p10 = pl.pallas_call(_proj_body, out_shape=out_shapes)(
        et2, eq2, eg2, emb_int, emb_tt, wt2, wq2, wg2,
        comb_W[0:21], comb_W[99:104])
    return (pt2.reshape(NT, HD), pq2.reshape(NQ, HD), pg2.reshape(NG, HD),
            p3, p10)


def _issue_gathers(pt_hbm, pq_hbm, pg_hbm, ti_v, qi_v, gi_v, tmp, i, b, sem):
    sl = pl.ds(i * CH, CH)
    pltpu.async_copy(pt_hbm.at[ti_v.at[sl]], tmp.at[b, 0], sem)
    pltpu.async_copy(pq_hbm.at[qi_v.at[sl]], tmp.at[b, 1], sem)
    pltpu.async_copy(pg_hbm.at[gi_v.at[sl]], tmp.at[b, 2], sem)


def _drain_gathers(pt_hbm, tmp, b, sem):
    for j in range(3):
        pltpu.make_async_copy(pt_hbm.at[pl.ds(0, CH)], tmp.at[b, j], sem).wait()


def _gather_body(tidx_hbm, qidx_hbm, gidx_hbm, pt_hbm, pq_hbm, pg_hbm,
                 ident_hbm, out_hbm,
                 ti_v, qi_v, gi_v, ident_v, tmp, acc_sh, sem0, sem1):
    c = lax.axis_index("c")
    s = lax.axis_index("s")
    wid = s * NC + c
    base = wid * PER_W

    pltpu.sync_copy(tidx_hbm.at[pl.ds(base, PER_W)], ti_v)
    pltpu.sync_copy(qidx_hbm.at[pl.ds(base, PER_W)], qi_v)
    pltpu.sync_copy(gidx_hbm.at[pl.ds(base, PER_W)], gi_v)
    pltpu.sync_copy(ident_hbm, ident_v)
    off = s * CH
    for k in range(CH // 16):
        sl = pl.ds(k * 16, 16)
        ident_v[sl] = ident_v[sl] + off

    _issue_gathers(pt_hbm, pq_hbm, pg_hbm, ti_v, qi_v, gi_v, tmp, 0, 0, sem0)

    sems = (sem0, sem1)

    def chunk(i, b):
        @pl.when(i < NCHUNK - 1)
        def _():
            _issue_gathers(pt_hbm, pq_hbm, pg_hbm, ti_v, qi_v, gi_v, tmp,
                           i + 1, 1 - b, sems[1 - b])
        _drain_gathers(pt_hbm, tmp, b, sems[b])
        mine = acc_sh.at[pl.ds(s * CH, CH)]
        pltpu.sync_copy(tmp.at[b, 0], mine)
        pltpu.sync_copy(tmp.at[b, 1], acc_sh.at[ident_v], add=True)
        pltpu.sync_copy(tmp.at[b, 2], acc_sh.at[ident_v], add=True)
        pltpu.sync_copy(mine, out_hbm.at[pl.ds(base + i * CH, CH),
                                         pl.ds(0, HD)])

    def two_chunks(k, carry):
        chunk(2 * k, 0)
        chunk(2 * k + 1, 1)
        return carry

    lax.fori_loop(0, NCHUNK // 2, two_chunks, 0)
    if NCHUNK % 2:
        chunk(NCHUNK - 1, 0)


def _gather_sum(tidx, qidx, gidx, pt, pq, pg):
    mesh = plsc.VectorSubcoreMesh(core_axis_name="c", subcore_axis_name="s")
    ident = jnp.arange(CH, dtype=jnp.int32)
    fn = pl.kernel(
        _gather_body,
        out_type=jax.ShapeDtypeStruct((NPOS, 128), _f32),
        mesh=mesh,
        compiler_params=pltpu.CompilerParams(use_tc_tiling_on_sc=False),
        scratch_types=[
            pltpu.VMEM((PER_W,), jnp.int32),
            pltpu.VMEM((PER_W,), jnp.int32),
            pltpu.VMEM((PER_W,), jnp.int32),
            pltpu.VMEM((CH,), jnp.int32),
            pltpu.VMEM((2, 3, CH, HD), _f32),
            pltpu.VMEM_SHARED((NS * CH, HD), _f32),
            pltpu.SemaphoreType.DMA,
            pltpu.SemaphoreType.DMA,
        ],
    )
    return fn(tidx, qidx, gidx, pt, pq, pg, ident)


CHC = 4096


def _dense_body(*refs):
    (f0, f1, f2, f3, f4, f5, f6, f7, f8, f9, f10, f11, f12, f13,
     gsum_ref, g_ref, h_ref, w60_ref, p3_ref, p10_ref, b_ref, out_ref) = refs
    feats = (f0, f1, f2, f3, f4, f5, f6, f7, f8, f9, f10, f11, f12, f13)
    xt = jnp.concatenate([f[...].reshape(1, CHC) for f in feats], axis=0)
    a = jax.nn.sigmoid(
        lax.dot_general(xt, g_ref[...], _DNT, preferred_element_type=_f32)
        + h_ref[...])
    o = _dot(a, w60_ref[...]) + b_ref[...] + gsum_ref[:, 0:HD]
    iv = xt[12:13, :]
    tv = xt[13:14, :]
    i3 = lax.broadcasted_iota(jnp.int32, (3, CHC), 0).astype(_f32)
    i10 = lax.broadcasted_iota(jnp.int32, (10, CHC), 0).astype(_f32)
    oh3 = jnp.where(iv == i3, 1.0, 0.0)
    oh10 = jnp.where(tv == i10, 1.0, 0.0)
    o = o + lax.dot_general(oh3, p3_ref[...], _DNT,
                            preferred_element_type=_f32)
    o = o + lax.dot_general(oh10, p10_ref[...], _DNT,
                            preferred_element_type=_f32)
    out_ref[...] = o


def _dense_part(feats, gsum2, G, h, W60, p3, p10, bias):
    grid = (NPOS // CHC,)
    fixed = lambda i: (0, 0)
    fspec = pl.BlockSpec((CHC,), lambda i: (i,))
    return pl.pallas_call(
        _dense_body,
        grid=grid,
        in_specs=[fspec] * 14 + [
            pl.BlockSpec((CHC, 128), lambda i: (i, 0)),
            pl.BlockSpec((14, 60), fixed),
            pl.BlockSpec((1, 60), fixed),
            pl.BlockSpec((60, HD), fixed),
            pl.BlockSpec((3, HD), fixed),
            pl.BlockSpec((10, HD), fixed),
            pl.BlockSpec((1, HD), fixed),
        ],
        out_specs=pl.BlockSpec((CHC, HD), lambda i: (i, 0)),
        out_shape=jax.ShapeDtypeStruct((NPOS, HD), _f32),
    )(*feats, gsum2, G, h, W60, p3, p10, bias)


def kernel(test, question, tag, correct, mask, interaction, dffclt, dscrmn,
           gussng, testTag, user_correct_answer, user_total_answer, user_acc,
           user_mean, assessment_mean, test_mean, knowledgeTag_mean,
           time_to_solve, prior_testTag_frequency, emb_interaction, emb_test,
           emb_question, emb_tag, emb_testTag, lin_W, lin_b, comb_W, comb_b):
    tidx = test.reshape(NPOS).astype(jnp.int32)
    qidx = question.reshape(NPOS).astype(jnp.int32)
    gidx = tag.reshape(NPOS).astype(jnp.int32)

    conti = [dffclt, dscrmn, gussng, user_correct_answer, user_total_answer,
             user_acc, user_mean, assessment_mean, test_mean,
             knowledgeTag_mean, time_to_solve, prior_testTag_frequency]
    feats = tuple(
        f.reshape(NPOS) for f in
        conti + [interaction.astype(_f32), testTag.astype(_f32)])

    G = jnp.concatenate(
        [jnp.kron(jnp.eye(12, dtype=_f32), lin_W[0:1]),
         jnp.zeros((2, 60), _f32)], axis=0)
    h = jnp.tile(lin_b, 12).reshape(1, 60)
    W60 = jnp.concatenate([comb_W[84:99], comb_W[104:149]], axis=0)

    pt, pq, pg, p3, p10 = _project_tables(
        emb_test, emb_question, emb_tag, emb_interaction, emb_testTag, comb_W)
    gsum = _gather_sum(tidx, qidx, gidx, pt, pq, pg)
    out = _dense_part(feats, gsum, G, h, W60, p3, p10, comb_b.reshape(1, HD))
    return out.reshape(B, S, HD)
